# baseline (speedup 1.0000x reference)
import jax
import jax.numpy as jnp
from jax import lax
from jax.experimental import pallas as pl
from jax.experimental.pallas import tpu as pltpu

_SRC_X = 2048
_SRC_Y = 2048
_ROWS = 4096
_COLS = 4096

_C = 1024
_NCH = _ROWS // _C
_D = 3
_ISRC = _SRC_X // _C
_LR = _SRC_X % _C
_LR8 = (_LR // 8) * 8


def _body(bt_ref, b_any, o_any, *rest):
    bufs = rest[:_D]
    in_sems = rest[_D:2 * _D]
    out_sems = rest[2 * _D:]

    def in_copy(i, d):
        return pltpu.make_async_copy(
            b_any.at[pl.ds(i * _C, _C), :], bufs[d], in_sems[d])

    def out_copy(i, d):
        return pltpu.make_async_copy(
            bufs[d], o_any.at[pl.ds(i * _C, _C), :], out_sems[d])

    for i in range(_D):
        in_copy(i, i).start()

    for i in range(_NCH):
        d = i % _D
        in_copy(i, d).wait()
        if i == _ISRC:
            ri = lax.broadcasted_iota(jnp.int32, (8, 128), 0)
            ci = lax.broadcasted_iota(jnp.int32, (8, 128), 1)
            sub = bufs[d][pl.ds(_LR8, 8), pl.ds(_SRC_Y, 128)]
            bufs[d][pl.ds(_LR8, 8), pl.ds(_SRC_Y, 128)] = jnp.where(
                (ri == _LR - _LR8) & (ci == 0), bt_ref[0, 0], sub)
        out_copy(i, d).start()
        nxt = i + _D
        if nxt < _NCH:
            out_copy(i, d).wait()
            in_copy(nxt, d).start()

    for i in range(_NCH - _D, _NCH):
        out_copy(i, i % _D).wait()


@jax.jit
def _scatter_copy(bt, b2d):
    return pl.pallas_call(
        _body,
        in_specs=[
            pl.BlockSpec(memory_space=pltpu.SMEM),
            pl.BlockSpec(memory_space=pl.ANY),
        ],
        out_specs=pl.BlockSpec(memory_space=pl.ANY),
        out_shape=jax.ShapeDtypeStruct((_ROWS, _COLS), jnp.float32),
        scratch_shapes=(
            [pltpu.VMEM((_C, _COLS), jnp.float32) for _ in range(_D)]
            + [pltpu.SemaphoreType.DMA for _ in range(2 * _D)]
        ),
    )(bt, b2d)


def kernel(B, Bt):
    out = _scatter_copy(Bt, B.reshape(_ROWS, _COLS))
    return out.reshape(B.shape)
